# Initial kernel scaffold; baseline (speedup 1.0000x reference)
import jax
import jax.numpy as jnp
from jax.experimental import pallas as pl


def kernel(x, edge_label_index):
    raise NotImplementedError("write your pallas kernel here")


# speedup vs baseline: 2.7983x; 2.7983x over previous
import jax
import jax.numpy as jnp
from jax import lax
from jax.experimental import pallas as pl
from jax.experimental.pallas import tpu as pltpu
from jax.experimental.pallas import tpu_sc as plsc

L = 16
NC, NS = 2, 16
NW = NC * NS
E = 320000
D = 128
EPW = E // NW
C = 80
NCHUNK = EPW // C
G = C // L


_DNUMS = lax.GatherDimensionNumbers(
    offset_dims=(), collapsed_slice_dims=(0,), start_index_map=(0,))


def _xlane_take(v, idx):
    return lax.gather(v, idx[:, None], _DNUMS, slice_sizes=(1,),
                      mode=lax.GatherScatterMode.PROMISE_IN_BOUNDS)


def _body(x_hbm, src_hbm, dst_hbm, out_hbm,
          idx_a, idx_b, rows_a, rows_b, out_v, sem_a, sem_b):
    wid = lax.axis_index("s") * NC + lax.axis_index("c")
    base = wid * EPW
    lane = lax.iota(jnp.int32, L)

    def chunk(i, carry):
        off = base + i * C
        pltpu.sync_copy(src_hbm.at[pl.ds(off, C)], idx_a)
        pltpu.sync_copy(dst_hbm.at[pl.ds(off, C)], idx_b)
        ca = pltpu.async_copy(x_hbm.at[idx_a], rows_a, sem_a)
        cb = pltpu.async_copy(x_hbm.at[idx_b], rows_b, sem_b)
        ca.wait()
        cb.wait()

        def group(g, gcarry):
            res = jnp.zeros((L,), jnp.float32)
            for j in range(L):
                e = g * L + j
                acc = rows_a[e, pl.ds(0, L)] * rows_b[e, pl.ds(0, L)]
                for k in range(1, D // L):
                    acc = acc + (rows_a[e, pl.ds(k * L, L)]
                                 * rows_b[e, pl.ds(k * L, L)])
                for sh in (8, 4, 2, 1):
                    acc = acc + _xlane_take(acc, lane ^ sh)
                res = jnp.where(lane == j, acc, res)
            out_v[pl.ds(g * L, L)] = res
            return gcarry

        lax.fori_loop(0, G, group, 0)
        pltpu.sync_copy(out_v, out_hbm.at[pl.ds(off, C)])
        return carry

    lax.fori_loop(0, NCHUNK, chunk, 0)


def kernel(x, edge_label_index):
    idx = edge_label_index.astype(jnp.int32)
    f = pl.kernel(
        _body,
        out_type=jax.ShapeDtypeStruct((E,), jnp.float32),
        mesh=plsc.VectorSubcoreMesh(core_axis_name="c", subcore_axis_name="s"),
        scratch_types=[
            pltpu.VMEM((C,), jnp.int32),
            pltpu.VMEM((C,), jnp.int32),
            pltpu.VMEM((C, D), jnp.float32),
            pltpu.VMEM((C, D), jnp.float32),
            pltpu.VMEM((C,), jnp.float32),
            pltpu.SemaphoreType.DMA,
            pltpu.SemaphoreType.DMA,
        ],
    )
    return f(x, idx[0], idx[1])


# speedup vs baseline: 3.7213x; 1.3298x over previous
import jax
import jax.numpy as jnp
from jax import lax
from jax.experimental import pallas as pl
from jax.experimental.pallas import tpu as pltpu
from jax.experimental.pallas import tpu_sc as plsc

L = 16
NC, NS = 2, 16
NW = NC * NS
E = 320000
D = 128
EPW = E // NW
C = 80
NCHUNK = EPW // C
G = C // L
NBUF = 2

_DNUMS = lax.GatherDimensionNumbers(
    offset_dims=(), collapsed_slice_dims=(0,), start_index_map=(0,))


def _xlane_take(v, idx):
    return lax.gather(v, idx[:, None], _DNUMS, slice_sizes=(1,),
                      mode=lax.GatherScatterMode.PROMISE_IN_BOUNDS)


def _body(x_hbm, src_hbm, dst_hbm, out_hbm,
          idx_a, idx_b, out_all,
          rows_a0, rows_b0, rows_a1, rows_b1,
          sem_a0, sem_b0, sem_a1, sem_b1):
    wid = lax.axis_index("s") * NC + lax.axis_index("c")
    base = wid * EPW
    lane = lax.iota(jnp.int32, L)

    pltpu.sync_copy(src_hbm.at[pl.ds(base, EPW)], idx_a)
    pltpu.sync_copy(dst_hbm.at[pl.ds(base, EPW)], idx_b)

    bufs = ((rows_a0, rows_b0, sem_a0, sem_b0),
            (rows_a1, rows_b1, sem_a1, sem_b1))

    def fire(cid, buf):
        ra, rb, sa, sb = buf
        off = jnp.minimum(cid, NCHUNK - 1) * C
        pltpu.async_copy(x_hbm.at[idx_a.at[pl.ds(off, C)]], ra, sa)
        pltpu.async_copy(x_hbm.at[idx_b.at[pl.ds(off, C)]], rb, sb)

    def drain(buf):
        ra, rb, sa, sb = buf
        pltpu.make_async_copy(x_hbm.at[idx_a.at[pl.ds(0, C)]], ra, sa).wait()
        pltpu.make_async_copy(x_hbm.at[idx_b.at[pl.ds(0, C)]], rb, sb).wait()

    def compute(cid, buf):
        ra, rb, _, _ = buf
        off = jnp.minimum(cid, NCHUNK - 1) * C
        for g in range(G):
            res = jnp.zeros((L,), jnp.float32)
            for j in range(L):
                e = g * L + j
                acc = ra[e, pl.ds(0, L)] * rb[e, pl.ds(0, L)]
                for k in range(1, D // L):
                    acc = acc + ra[e, pl.ds(k * L, L)] * rb[e, pl.ds(k * L, L)]
                for sh in (8, 4, 2, 1):
                    acc = acc + _xlane_take(acc, lane ^ sh)
                res = jnp.where(lane == j, acc, res)
            out_all[pl.ds(off + g * L, L)] = res

    for b in range(NBUF):
        fire(b, bufs[b])

    def pair(p, carry):
        i = p * NBUF
        for b in range(NBUF):
            drain(bufs[b])
            compute(i + b, bufs[b])
            fire(i + b + NBUF, bufs[b])
        return carry

    lax.fori_loop(0, (NCHUNK + NBUF - 1) // NBUF, pair, 0)

    for b in range(NBUF):
        drain(bufs[b])

    pltpu.sync_copy(out_all, out_hbm.at[pl.ds(base, EPW)])


def kernel(x, edge_label_index):
    idx = edge_label_index.astype(jnp.int32)
    f = pl.kernel(
        _body,
        out_type=jax.ShapeDtypeStruct((E,), jnp.float32),
        mesh=plsc.VectorSubcoreMesh(core_axis_name="c", subcore_axis_name="s"),
        scratch_types=[
            pltpu.VMEM((EPW,), jnp.int32),
            pltpu.VMEM((EPW,), jnp.int32),
            pltpu.VMEM((EPW,), jnp.float32),
            pltpu.VMEM((C, D), jnp.float32),
            pltpu.VMEM((C, D), jnp.float32),
            pltpu.VMEM((C, D), jnp.float32),
            pltpu.VMEM((C, D), jnp.float32),
            pltpu.SemaphoreType.DMA,
            pltpu.SemaphoreType.DMA,
            pltpu.SemaphoreType.DMA,
            pltpu.SemaphoreType.DMA,
        ],
    )
    return f(x, idx[0], idx[1])
